# baseline (speedup 1.0000x reference)
import dataclasses
import functools

import jax
import jax.numpy as jnp
from jax import lax
from jax.experimental import pallas as pl
from jax.experimental.pallas import tpu as pltpu
from jax.experimental.pallas import tpu_sc as plsc

R = 16384
C = 1000
B = 200
NC, NS, L = 2, 16, 16
NW = NC * NS
HALF = R // 2
ROWS_PER_W = HALF // NW
BLK = 32
NBLK = ROWS_PER_W // BLK
STR = 128
BPS = STR // BLK


def _compiler_params():
    cp = pltpu.CompilerParams()
    fields = pltpu.CompilerParams.__dataclass_fields__
    if "needs_layout_passes" in fields:
        cp = dataclasses.replace(cp, needs_layout_passes=False)
    if "disable_bounds_checks" in fields:
        cp = dataclasses.replace(cp, disable_bounds_checks=True)
    return cp


def _make_call(out_cols, col_base, forward_prev):
    mesh = plsc.VectorSubcoreMesh(core_axis_name="c", subcore_axis_name="s")
    scratch = [
        pltpu.VMEM((2, BLK, C), jnp.float32),
        pltpu.VMEM((B, STR), jnp.int32),
        pltpu.VMEM((B, STR), jnp.float32),
        pltpu.SemaphoreType.DMA((2,)),
    ]
    if forward_prev:
        scratch.append(pltpu.SemaphoreType.DMA)

    @functools.partial(
        pl.kernel,
        compiler_params=_compiler_params(),
        out_type=jax.ShapeDtypeStruct((B, out_cols), jnp.float32),
        mesh=mesh,
        scratch_types=scratch,
    )
    def k(*refs):
        if forward_prev:
            tbl_hbm, idx_hbm, prev_hbm, out_hbm, rows_v, idx_v, out_v, st_, sp_ = refs
        else:
            tbl_hbm, idx_hbm, out_hbm, rows_v, idx_v, out_v, st_ = refs
            prev_hbm = sp_ = None
        wid = lax.axis_index("s") * NC + lax.axis_index("c")
        lrow0 = wid * ROWS_PER_W
        orow0 = col_base + lrow0

        if forward_prev:
            fwd = pltpu.make_async_copy(
                prev_hbm.at[:, pl.ds(lrow0, ROWS_PER_W)],
                out_hbm.at[:, pl.ds(lrow0, ROWS_PER_W)], sp_)
            fwd.start()

        rowvec = [lax.iota(jnp.int32, L) + ic * L for ic in range(BLK // L)]

        def tbl_copy(g, bslot):
            return pltpu.make_async_copy(
                tbl_hbm.at[pl.ds(lrow0 + g * BLK, BLK)],
                rows_v.at[bslot], st_.at[bslot])

        tbl_copy(0, 0).start()

        @pl.loop(0, NBLK)
        def _(g):
            b = lax.rem(g, 2)
            tbrel = lax.rem(g, BPS)
            stripe0 = orow0 + (g // BPS) * STR

            @pl.when(tbrel == 0)
            def _():
                pltpu.sync_copy(idx_hbm.at[:, pl.ds(stripe0, STR)], idx_v)

            @pl.when(g + 1 < NBLK)
            def _():
                tbl_copy(g + 1, 1 - b).start()

            tbl_copy(g, b).wait()
            rows_b = rows_v.at[b]

            @pl.loop(0, B, step=2)
            def _(j):
                work = []
                for dj in (0, 1):
                    for ic in range(BLK // L):
                        o = tbrel * BLK + ic * L
                        work.append(
                            (dj, o, rowvec[ic], idx_v[j + dj, pl.ds(o, L)]))
                vals = [(dj, o, plsc.load_gather(rows_b, [rv, col]))
                        for (dj, o, rv, col) in work]
                for (dj, o, v) in vals:
                    out_v[j + dj, pl.ds(o, L)] = v

            @pl.when(tbrel == BPS - 1)
            def _():
                pltpu.sync_copy(out_v, out_hbm.at[:, pl.ds(stripe0, STR)])

        if forward_prev:
            fwd.wait()

    return k


def kernel(input1, input2):
    idx_t = input2.astype(jnp.int32).T
    k0 = _make_call(HALF, 0, False)
    k1 = _make_call(R, HALF, True)
    o0 = k0(input1[:HALF], idx_t)
    out_t = k1(input1[HALF:], idx_t, o0)
    return out_t.T


# speedup vs baseline: 2.8299x; 2.8299x over previous
import dataclasses
import functools

import jax
import jax.numpy as jnp
from jax import lax
from jax.experimental import pallas as pl
from jax.experimental.pallas import tpu as pltpu
from jax.experimental.pallas import tpu_sc as plsc

R = 16384
C = 1000
B = 200
NC, NS, L = 2, 16, 16
NW = NC * NS
ROWS_PER_W = R // NW
BLK = 32
NBLK = ROWS_PER_W // BLK
STR = 128
BPS = STR // BLK


def kernel(input1, input2):
    idx_t = input2.astype(jnp.int32).T
    mesh = plsc.VectorSubcoreMesh(core_axis_name="c", subcore_axis_name="s")
    cp = pltpu.CompilerParams()
    fields = pltpu.CompilerParams.__dataclass_fields__
    if "needs_layout_passes" in fields:
        cp = dataclasses.replace(cp, needs_layout_passes=False)
    if "disable_bounds_checks" in fields:
        cp = dataclasses.replace(cp, disable_bounds_checks=True)

    @functools.partial(
        pl.kernel,
        compiler_params=cp,
        out_type=jax.ShapeDtypeStruct((B, R), jnp.float32),
        mesh=mesh,
        scratch_types=[
            pltpu.VMEM((2, BLK, C), jnp.float32),
            pltpu.VMEM((B, STR), jnp.int32),
            pltpu.VMEM((B, STR), jnp.float32),
            pltpu.SemaphoreType.DMA((2,)),
        ],
    )
    def k(tbl_hbm, idx_hbm, out_hbm, rows_v, idx_v, out_v, st_):
        wid = lax.axis_index("s") * NC + lax.axis_index("c")
        row0 = wid * ROWS_PER_W

        rowvec = [lax.iota(jnp.int32, L) + ic * L for ic in range(BLK // L)]

        def tbl_copy(g, bslot):
            return pltpu.make_async_copy(
                tbl_hbm.at[pl.ds(row0 + g * BLK, BLK)],
                rows_v.at[bslot], st_.at[bslot])

        tbl_copy(0, 0).start()

        @pl.loop(0, NBLK)
        def _(g):
            b = lax.rem(g, 2)
            tbrel = lax.rem(g, BPS)
            stripe0 = row0 + (g // BPS) * STR

            @pl.when(tbrel == 0)
            def _():
                pltpu.sync_copy(idx_hbm.at[:, pl.ds(stripe0, STR)], idx_v)

            @pl.when(g + 1 < NBLK)
            def _():
                tbl_copy(g + 1, 1 - b).start()

            tbl_copy(g, b).wait()
            rows_b = rows_v.at[b]

            @pl.loop(0, B, step=2)
            def _(j):
                work = []
                for dj in (0, 1):
                    for ic in range(BLK // L):
                        o = tbrel * BLK + ic * L
                        work.append(
                            (dj, o, rowvec[ic], idx_v[j + dj, pl.ds(o, L)]))
                vals = [(dj, o, plsc.load_gather(rows_b, [rv, col]))
                        for (dj, o, rv, col) in work]
                for (dj, o, v) in vals:
                    out_v[j + dj, pl.ds(o, L)] = v

            @pl.when(tbrel == BPS - 1)
            def _():
                pltpu.sync_copy(out_v, out_hbm.at[:, pl.ds(stripe0, STR)])

    out_t = k(input1, idx_t)
    return out_t.T
